# baseline (speedup 1.0000x reference)
import functools

import jax
import jax.numpy as jnp
from jax import lax
from jax.experimental import pallas as pl
from jax.experimental.pallas import tpu as pltpu
from jax.experimental.pallas import tpu_sc as plsc


def _mmoe_tc_body(x_ref, we_ref, be_ref, wg_ref, wt_ref, s_ref, g_ref):
    e = pl.program_id(0)
    j = pl.program_id(1)

    @pl.when(jnp.logical_and(e == 0, j == 0))
    def _init():
        gm = jnp.dot(x_ref[...], wg_ref[...])
        g_ref[...] = gm.T

    h = jnp.maximum(jnp.dot(x_ref[...], we_ref[0]) + be_ref[0], 0.0)
    p = jnp.dot(h, wt_ref[...])

    @pl.when(j == 0)
    def _first():
        s_ref[0] = p.T

    @pl.when(j != 0)
    def _rest():
        s_ref[0] += p.T


def _mmoe_sc_body(g_hbm, s_hbm, bt_hbm, out_hbm, g_v, s_v, bt_v, out_v):
    E = s_v.shape[0]
    T = s_v.shape[1]
    RW = g_v.shape[1]
    cid = lax.axis_index("c")
    sid = lax.axis_index("s")
    base = sid * RW

    @pl.when(cid == 0)
    def _work():
        pltpu.sync_copy(g_hbm.at[:, pl.ds(base, RW)], g_v)
        pltpu.sync_copy(s_hbm.at[:, :, pl.ds(base, RW)], s_v)
        pltpu.sync_copy(bt_hbm, bt_v)

        for c in range(RW // 16):
            sl = pl.ds(c * 16, 16)
            for t in range(T):
                g = [g_v[t * E + e2, sl] for e2 in range(E)]
                m = g[0]
                for e2 in range(1, E):
                    m = jnp.maximum(m, g[e2])
                ex = jnp.exp(g[0] - m)
                num = ex * s_v[0, t, sl]
                den = ex
                for e2 in range(1, E):
                    ex = jnp.exp(g[e2] - m)
                    num = num + ex * s_v[e2, t, sl]
                    den = den + ex
                out_v[t, sl] = num / den + bt_v[t, pl.ds(0, 16)]

        pltpu.sync_copy(out_v, out_hbm.at[:, pl.ds(base, RW)])


def kernel(x, We, be, Wg, Wt, bt):
    N, D = x.shape
    E, _, F = We.shape
    T = Wg.shape[0]
    bf = min(1024, F)
    J = F // bf

    wg_flat = jnp.transpose(Wg, (1, 0, 2)).reshape(D, T * E)
    wt_flat = jnp.transpose(Wt[:, :, 0], (1, 0))
    be3 = be.reshape(E, 1, F)

    s_et, g_logits = pl.pallas_call(
        _mmoe_tc_body,
        grid=(E, J),
        in_specs=[
            pl.BlockSpec((N, D), lambda e, j: (0, 0)),
            pl.BlockSpec((1, D, bf), lambda e, j: (e, 0, j)),
            pl.BlockSpec((1, 1, bf), lambda e, j: (e, 0, j)),
            pl.BlockSpec((D, T * E), lambda e, j: (0, 0)),
            pl.BlockSpec((bf, T), lambda e, j: (j, 0)),
        ],
        out_specs=[
            pl.BlockSpec((1, T, N), lambda e, j: (e, 0, 0)),
            pl.BlockSpec((T * E, N), lambda e, j: (0, 0)),
        ],
        out_shape=[
            jax.ShapeDtypeStruct((E, T, N), jnp.float32),
            jax.ShapeDtypeStruct((T * E, N), jnp.float32),
        ],
        compiler_params=pltpu.CompilerParams(
            dimension_semantics=("arbitrary", "arbitrary"),
        ),
    )(x, We, be3, wg_flat, wt_flat)

    info = plsc.get_sparse_core_info()
    rw = N // info.num_subcores
    bt_b = jnp.broadcast_to(bt.reshape(T, 1), (T, 16))

    sc_combine = functools.partial(
        pl.kernel,
        mesh=plsc.VectorSubcoreMesh(core_axis_name="c", subcore_axis_name="s"),
        out_type=jax.ShapeDtypeStruct((T, N), jnp.float32),
        scratch_types=[
            pltpu.VMEM((T * E, rw), jnp.float32),
            pltpu.VMEM((E, T, rw), jnp.float32),
            pltpu.VMEM((T, 16), jnp.float32),
            pltpu.VMEM((T, rw), jnp.float32),
        ],
    )(_mmoe_sc_body)

    out_tn = sc_combine(g_logits, s_et, bt_b)
    return out_tn[:, :, None]


# speedup vs baseline: 1.0066x; 1.0066x over previous
import functools

import jax
import jax.numpy as jnp
from jax import lax
from jax.experimental import pallas as pl
from jax.experimental.pallas import tpu as pltpu
from jax.experimental.pallas import tpu_sc as plsc


def _mmoe_tc_body(x_ref, we_ref, be_ref, wg_ref, wt_ref, bt_ref, s_ref, g_ref,
                  acc_ref):
    e = pl.program_id(0)
    j = pl.program_id(1)
    nj = pl.num_programs(1)

    @pl.when(jnp.logical_and(e == 0, j == 0))
    def _init():
        gm = jnp.dot(x_ref[...], wg_ref[...])
        g_ref[...] = gm.T

    h = jnp.maximum(jnp.dot(x_ref[...], we_ref[0]) + be_ref[0], 0.0)
    p = jnp.dot(h, wt_ref[...])

    @pl.when(j == 0)
    def _first():
        acc_ref[...] = p + bt_ref[...]

    @pl.when(j != 0)
    def _rest():
        acc_ref[...] += p

    @pl.when(j == nj - 1)
    def _emit():
        s_ref[0] = acc_ref[...].T


def _mmoe_sc_body(g_hbm, s_hbm, out_hbm, g_v, s_v, out_v):
    E = s_v.shape[0]
    T = s_v.shape[1]
    RW = g_v.shape[1]
    cid = lax.axis_index("c")
    sid = lax.axis_index("s")
    base = sid * RW

    @pl.when(cid == 0)
    def _work():
        pltpu.sync_copy(g_hbm.at[:, pl.ds(base, RW)], g_v)
        pltpu.sync_copy(s_hbm.at[:, :, pl.ds(base, RW)], s_v)

        for c in range(RW // 16):
            sl = pl.ds(c * 16, 16)
            for t in range(T):
                g = [g_v[t * E + e2, sl] for e2 in range(E)]
                m = g[0]
                for e2 in range(1, E):
                    m = jnp.maximum(m, g[e2])
                ex = jnp.exp(g[0] - m)
                num = ex * s_v[0, t, sl]
                den = ex
                for e2 in range(1, E):
                    ex = jnp.exp(g[e2] - m)
                    num = num + ex * s_v[e2, t, sl]
                    den = den + ex
                out_v[t, sl] = num / den

        pltpu.sync_copy(out_v, out_hbm.at[:, pl.ds(base, RW)])


def kernel(x, We, be, Wg, Wt, bt):
    N, D = x.shape
    E, _, F = We.shape
    T = Wg.shape[0]
    bf = min(1024, F)
    J = F // bf

    wg_flat = jnp.transpose(Wg, (1, 0, 2)).reshape(D, T * E)
    wt_flat = jnp.transpose(Wt[:, :, 0], (1, 0))
    be3 = be.reshape(E, 1, F)
    bt_row = bt.reshape(1, T)

    s_et, g_logits = pl.pallas_call(
        _mmoe_tc_body,
        grid=(E, J),
        in_specs=[
            pl.BlockSpec((N, D), lambda e, j: (0, 0)),
            pl.BlockSpec((1, D, bf), lambda e, j: (e, 0, j)),
            pl.BlockSpec((1, 1, bf), lambda e, j: (e, 0, j)),
            pl.BlockSpec((D, T * E), lambda e, j: (0, 0)),
            pl.BlockSpec((bf, T), lambda e, j: (j, 0)),
            pl.BlockSpec((1, T), lambda e, j: (0, 0)),
        ],
        out_specs=[
            pl.BlockSpec((1, T, N), lambda e, j: (e, 0, 0)),
            pl.BlockSpec((T * E, N), lambda e, j: (0, 0)),
        ],
        out_shape=[
            jax.ShapeDtypeStruct((E, T, N), jnp.float32),
            jax.ShapeDtypeStruct((T * E, N), jnp.float32),
        ],
        scratch_shapes=[pltpu.VMEM((N, T), jnp.float32)],
        compiler_params=pltpu.CompilerParams(
            dimension_semantics=("arbitrary", "arbitrary"),
        ),
    )(x, We, be3, wg_flat, wt_flat, bt_row)

    info = plsc.get_sparse_core_info()
    rw = N // info.num_subcores

    sc_combine = functools.partial(
        pl.kernel,
        mesh=plsc.VectorSubcoreMesh(core_axis_name="c", subcore_axis_name="s"),
        out_type=jax.ShapeDtypeStruct((T, N), jnp.float32),
        scratch_types=[
            pltpu.VMEM((T * E, rw), jnp.float32),
            pltpu.VMEM((E, T, rw), jnp.float32),
            pltpu.VMEM((T, rw), jnp.float32),
        ],
    )(_mmoe_sc_body)

    out_tn = sc_combine(g_logits, s_et)
    return out_tn[:, :, None]
